# baseline (device time: 72505 ns/iter reference)
import numpy as np
import jax
import jax.numpy as jnp
from jax import lax
from jax.experimental import pallas as pl
from jax.experimental.pallas import tpu as pltpu

N_DEV = 4
B_LOC = 2
SQ = 256
D = 768
HQ_LOC = 4
DH = 64
HD_LOC = HQ_LOC * DH


def _rope_tables():
    inv = 1.0 / (10000.0 ** (np.arange(0, DH, 2) / DH))
    pos = np.arange(SQ)[:, None] * inv[None, :]
    cos = np.repeat(np.cos(pos), 2, axis=-1).astype(np.float32)
    sin = np.repeat(np.sin(pos), 2, axis=-1).astype(np.float32)
    cosf = np.tile(cos, (1, HQ_LOC))
    sinf = np.tile(sin, (1, HQ_LOC))
    p1 = np.zeros((DH, DH), np.float32)
    for i in range(DH // 2):
        p1[2 * i + 1, 2 * i] = -1.0
        p1[2 * i, 2 * i + 1] = 1.0
    P = np.kron(np.eye(HQ_LOC, dtype=np.float32), p1)
    return cosf, sinf, P


_COSF, _SINF, _P = _rope_tables()


def _body(x_ref, wq_ref, wk_ref, wv_ref, wo_ref, cos_ref, sin_ref, p_ref,
          out_ref,
          x_bf, qkv_comm, wo_comm, ctx_buf,
          qkv_send, qkv_recv, wo_send, wo_recv):
    my = lax.axis_index("i")
    left = lax.rem(my + N_DEV - 1, N_DEV)
    right = lax.rem(my + 1, N_DEV)

    barrier = pltpu.get_barrier_semaphore()
    for nbr in (left, right):
        pl.semaphore_signal(barrier, inc=1, device_id=(nbr,),
                            device_id_type=pl.DeviceIdType.MESH)
    pl.semaphore_wait(barrier, 2)

    x_bf[...] = x_ref[...].astype(jnp.bfloat16)
    qkv_comm[0, 0] = wq_ref[...].astype(jnp.bfloat16)
    qkv_comm[0, 1] = wk_ref[...].astype(jnp.bfloat16)
    qkv_comm[0, 2] = wv_ref[...].astype(jnp.bfloat16)
    wo_comm[0] = wo_ref[...].astype(jnp.bfloat16)

    cosf = cos_ref[...]
    sinf = sin_ref[...]
    P = p_ref[...]

    for h in range(N_DEV):
        if h < N_DEV - 1:
            rq = pltpu.make_async_remote_copy(
                src_ref=qkv_comm.at[h], dst_ref=qkv_comm.at[h + 1],
                send_sem=qkv_send.at[h], recv_sem=qkv_recv.at[h],
                device_id=(right,), device_id_type=pl.DeviceIdType.MESH)
            ro = pltpu.make_async_remote_copy(
                src_ref=wo_comm.at[h], dst_ref=wo_comm.at[h + 1],
                send_sem=wo_send.at[h], recv_sem=wo_recv.at[h],
                device_id=(right,), device_id_type=pl.DeviceIdType.MESH)
            rq.start()
            ro.start()

        wq = qkv_comm[h, 0]
        wk = qkv_comm[h, 1]
        wv = qkv_comm[h, 2]
        wo = wo_comm[h]
        for b in range(B_LOC):
            xb = x_bf[b]
            q = jnp.dot(xb, wq, preferred_element_type=jnp.float32)
            k = jnp.dot(xb, wk, preferred_element_type=jnp.float32)
            v = jnp.dot(xb, wv,
                        preferred_element_type=jnp.float32).astype(jnp.bfloat16)
            qr = (q * cosf
                  + jnp.dot(q.astype(jnp.bfloat16), P,
                            preferred_element_type=jnp.float32) * sinf
                  ).astype(jnp.bfloat16)
            kr = (k * cosf
                  + jnp.dot(k.astype(jnp.bfloat16), P,
                            preferred_element_type=jnp.float32) * sinf
                  ).astype(jnp.bfloat16)
            for hh in range(HQ_LOC):
                sl = slice(hh * DH, (hh + 1) * DH)
                s = lax.dot_general(
                    qr[:, sl], kr[:, sl], (((1,), (1,)), ((), ())),
                    preferred_element_type=jnp.float32) * 0.125
                m = jnp.max(s, axis=1, keepdims=True)
                e = jnp.exp(s - m)
                w = e / jnp.sum(e, axis=1, keepdims=True)
                ctx_buf[:, sl] = jnp.dot(
                    w.astype(jnp.bfloat16), v[:, sl],
                    preferred_element_type=jnp.float32).astype(jnp.bfloat16)
            contrib = jnp.dot(ctx_buf[...], wo,
                              preferred_element_type=jnp.float32)
            if h == 0:
                out_ref[b] = contrib
            else:
                out_ref[b] = out_ref[b] + contrib

        if h < N_DEV - 1:
            rq.wait()
            ro.wait()


def kernel(x, Wq, Wk, Wv, Wo):
    cos_in = jnp.asarray(_COSF)
    sin_in = jnp.asarray(_SINF)
    p_in = jnp.asarray(_P, dtype=jnp.bfloat16)
    return pl.pallas_call(
        _body,
        out_shape=jax.ShapeDtypeStruct((B_LOC, SQ, D), jnp.float32),
        in_specs=[pl.BlockSpec(memory_space=pltpu.VMEM)] * 8,
        out_specs=pl.BlockSpec(memory_space=pltpu.VMEM),
        scratch_shapes=[
            pltpu.VMEM((B_LOC, SQ, D), jnp.bfloat16),
            pltpu.VMEM((N_DEV, 3, D, HD_LOC), jnp.bfloat16),
            pltpu.VMEM((N_DEV, HD_LOC, D), jnp.bfloat16),
            pltpu.VMEM((SQ, HD_LOC), jnp.bfloat16),
            pltpu.SemaphoreType.DMA((N_DEV - 1,)),
            pltpu.SemaphoreType.DMA((N_DEV - 1,)),
            pltpu.SemaphoreType.DMA((N_DEV - 1,)),
            pltpu.SemaphoreType.DMA((N_DEV - 1,)),
        ],
        compiler_params=pltpu.CompilerParams(collective_id=0),
    )(x, Wq, Wk, Wv, Wo, cos_in, sin_in, p_in)


# device time: 47366 ns/iter; 1.5307x vs baseline; 1.5307x over previous
import numpy as np
import jax
import jax.numpy as jnp
from jax import lax
from jax.experimental import pallas as pl
from jax.experimental.pallas import tpu as pltpu

N_DEV = 4
B_LOC = 2
SQ = 256
D = 768
DH = 64
HC = 2
CW = HC * DH
BS = B_LOC * SQ


def _rope_tables():
    inv = 1.0 / (10000.0 ** (np.arange(0, DH, 2) / DH))
    pos = np.arange(SQ)[:, None] * inv[None, :]
    cos = np.repeat(np.cos(pos), 2, axis=-1).astype(np.float32)
    sin = np.repeat(np.sin(pos), 2, axis=-1).astype(np.float32)
    cosf = np.tile(cos, (B_LOC, HC))
    sinf = np.tile(sin, (B_LOC, HC))
    p1 = np.zeros((DH, DH), np.float32)
    for i in range(DH // 2):
        p1[2 * i + 1, 2 * i] = -1.0
        p1[2 * i, 2 * i + 1] = 1.0
    P = np.kron(np.eye(HC, dtype=np.float32), p1)
    return cosf, sinf, P


_COSF, _SINF, _P = _rope_tables()


def _body(x_ref, wq_ref, wk_ref, wv_ref, wo_ref, cos_ref, sin_ref, p_ref,
          out_ref,
          x_bf, acc, ctx_buf, qkv_comm, wo_comm,
          qkv_send, qkv_recv, wo_send, wo_recv):
    my = lax.axis_index("i")
    left = lax.rem(my + N_DEV - 1, N_DEV)
    right = lax.rem(my + 1, N_DEV)

    barrier = pltpu.get_barrier_semaphore()
    for nbr in (left, right):
        pl.semaphore_signal(barrier, inc=1, device_id=(nbr,),
                            device_id_type=pl.DeviceIdType.MESH)
    pl.semaphore_wait(barrier, 2)

    x_bf[...] = x_ref[...].astype(jnp.bfloat16).reshape(BS, D)
    for r in range(2):
        csl = slice(r * CW, (r + 1) * CW)
        qkv_comm[r, 0, 0] = wq_ref[:, csl].astype(jnp.bfloat16)
        qkv_comm[r, 0, 1] = wk_ref[:, csl].astype(jnp.bfloat16)
        qkv_comm[r, 0, 2] = wv_ref[:, csl].astype(jnp.bfloat16)
        wo_comm[r, 0] = wo_ref[csl, :].astype(jnp.bfloat16)

    cosf = cos_ref[...]
    sinf = sin_ref[...]
    P = p_ref[...]
    x_flat = x_bf[...]

    for h in range(N_DEV):
        rdmas = []
        if h < N_DEV - 1:
            for r, dst in ((0, right), (1, left)):
                rq = pltpu.make_async_remote_copy(
                    src_ref=qkv_comm.at[r, h], dst_ref=qkv_comm.at[r, h + 1],
                    send_sem=qkv_send.at[r, h], recv_sem=qkv_recv.at[r, h],
                    device_id=(dst,), device_id_type=pl.DeviceIdType.MESH)
                ro = pltpu.make_async_remote_copy(
                    src_ref=wo_comm.at[r, h], dst_ref=wo_comm.at[r, h + 1],
                    send_sem=wo_send.at[r, h], recv_sem=wo_recv.at[r, h],
                    device_id=(dst,), device_id_type=pl.DeviceIdType.MESH)
                rq.start()
                ro.start()
                rdmas += [rq, ro]

        for r in range(2):
            wq = qkv_comm[r, h, 0]
            wk = qkv_comm[r, h, 1]
            wv = qkv_comm[r, h, 2]
            wo = wo_comm[r, h]
            q = jnp.dot(x_flat, wq, preferred_element_type=jnp.float32)
            k = jnp.dot(x_flat, wk, preferred_element_type=jnp.float32)
            v = jnp.dot(x_flat, wv,
                        preferred_element_type=jnp.float32).astype(jnp.bfloat16)
            qr = (q * cosf
                  + jnp.dot(q.astype(jnp.bfloat16), P,
                            preferred_element_type=jnp.float32) * sinf
                  ).astype(jnp.bfloat16)
            kr = (k * cosf
                  + jnp.dot(k.astype(jnp.bfloat16), P,
                            preferred_element_type=jnp.float32) * sinf
                  ).astype(jnp.bfloat16)
            for b in range(B_LOC):
                rsl = slice(b * SQ, (b + 1) * SQ)
                for hh in range(HC):
                    csl = slice(hh * DH, (hh + 1) * DH)
                    s = lax.dot_general(
                        qr[rsl, csl], kr[rsl, csl], (((1,), (1,)), ((), ())),
                        preferred_element_type=jnp.float32) * 0.125
                    m = jnp.max(s, axis=1, keepdims=True)
                    e = jnp.exp(s - m)
                    w = e / jnp.sum(e, axis=1, keepdims=True)
                    ctx_buf[rsl, csl] = jnp.dot(
                        w.astype(jnp.bfloat16), v[rsl, csl],
                        preferred_element_type=jnp.float32).astype(jnp.bfloat16)
            contrib = jnp.dot(ctx_buf[...], wo,
                              preferred_element_type=jnp.float32)
            if h == 0 and r == 0:
                acc[...] = contrib
            else:
                acc[...] = acc[...] + contrib

        for rdma in rdmas:
            rdma.wait()

    out_ref[...] = acc[...].reshape(B_LOC, SQ, D)


def kernel(x, Wq, Wk, Wv, Wo):
    cos_in = jnp.asarray(_COSF)
    sin_in = jnp.asarray(_SINF)
    p_in = jnp.asarray(_P, dtype=jnp.bfloat16)
    return pl.pallas_call(
        _body,
        out_shape=jax.ShapeDtypeStruct((B_LOC, SQ, D), jnp.float32),
        in_specs=[pl.BlockSpec(memory_space=pltpu.VMEM)] * 8,
        out_specs=pl.BlockSpec(memory_space=pltpu.VMEM),
        scratch_shapes=[
            pltpu.VMEM((BS, D), jnp.bfloat16),
            pltpu.VMEM((BS, D), jnp.float32),
            pltpu.VMEM((BS, CW), jnp.bfloat16),
            pltpu.VMEM((2, N_DEV, 3, D, CW), jnp.bfloat16),
            pltpu.VMEM((2, N_DEV, CW, D), jnp.bfloat16),
            pltpu.SemaphoreType.DMA((2, N_DEV - 1)),
            pltpu.SemaphoreType.DMA((2, N_DEV - 1)),
            pltpu.SemaphoreType.DMA((2, N_DEV - 1)),
            pltpu.SemaphoreType.DMA((2, N_DEV - 1)),
        ],
        compiler_params=pltpu.CompilerParams(collective_id=0),
    )(x, Wq, Wk, Wv, Wo, cos_in, sin_in, p_in)


# device time: 37584 ns/iter; 1.9291x vs baseline; 1.2603x over previous
import numpy as np
import jax
import jax.numpy as jnp
from jax import lax
from jax.experimental import pallas as pl
from jax.experimental.pallas import tpu as pltpu

N_DEV = 4
B_LOC = 2
SQ = 256
D = 768
DH = 64
HQ_LOC = 4
HC = 2
CW = HC * DH
PW = HQ_LOC * DH
BS = B_LOC * SQ


def _rope_tables():
    inv = 1.0 / (10000.0 ** (np.arange(0, DH, 2) / DH))
    pos = np.arange(SQ)[:, None] * inv[None, :]
    cos = np.repeat(np.cos(pos), 2, axis=-1).astype(np.float32)
    sin = np.repeat(np.sin(pos), 2, axis=-1).astype(np.float32)
    cosf = np.tile(cos, (B_LOC, HQ_LOC))
    sinf = np.tile(sin, (B_LOC, HQ_LOC))
    p1 = np.zeros((DH, DH), np.float32)
    for i in range(DH // 2):
        p1[2 * i + 1, 2 * i] = -1.0
        p1[2 * i, 2 * i + 1] = 1.0
    P = np.kron(np.eye(HQ_LOC, dtype=np.float32), p1)
    return cosf, sinf, P


_COSF, _SINF, _P = _rope_tables()


def _body(x_ref, wqt_ref, wkt_ref, wvt_ref, wo_ref, cos_ref, sin_ref, p_ref,
          out_ref,
          x_bf, ctx_buf, comm,
          send_sems, recv_sems):
    my = lax.axis_index("i")
    left = lax.rem(my + N_DEV - 1, N_DEV)
    right = lax.rem(my + 1, N_DEV)

    barrier = pltpu.get_barrier_semaphore()
    for nbr in (left, right):
        pl.semaphore_signal(barrier, inc=1, device_id=(nbr,),
                            device_id_type=pl.DeviceIdType.MESH)
    pl.semaphore_wait(barrier, 2)

    comm[0, 0, 0] = (wqt_ref[...] * 0.125).astype(jnp.bfloat16)
    comm[0, 0, 1] = wkt_ref[...].astype(jnp.bfloat16)

    cosf = cos_ref[...]
    sinf = sin_ref[...]
    P = p_ref[...]
    x_flat = x_bf[...]

    def make_rdma(r, h, s):
        dst = right if r == 0 else left
        return pltpu.make_async_remote_copy(
            src_ref=comm.at[h, s, :, pl.ds(r * CW, CW), :],
            dst_ref=comm.at[h + 1, s, :, pl.ds(r * CW, CW), :],
            send_sem=send_sems.at[r, h, s],
            recv_sem=recv_sems.at[r, h, s],
            device_id=(dst,), device_id_type=pl.DeviceIdType.MESH)

    started = []
    for r in range(2):
        rdma = make_rdma(r, 0, 0)
        rdma.start()
        started.append(rdma)
    comm[0, 1, 0] = wvt_ref[...].astype(jnp.bfloat16)
    comm[0, 1, 1] = wo_ref[...].astype(jnp.bfloat16)
    for r in range(2):
        rdma = make_rdma(r, 0, 1)
        rdma.start()
        started.append(rdma)
    x_bf[...] = x_ref[...].astype(jnp.bfloat16).reshape(BS, D)

    for h in range(N_DEV):
        if h > 0:
            for r in range(2):
                make_rdma(r, h - 1, 0).wait_recv()
        if 0 < h < N_DEV - 1:
            for r in range(2):
                rdma = make_rdma(r, h, 0)
                rdma.start()
                started.append(rdma)

        wqf = comm[h, 0, 0]
        wkf = comm[h, 0, 1]
        q = lax.dot_general(
            x_flat, wqf, (((1,), (1,)), ((), ())),
            preferred_element_type=jnp.float32).astype(jnp.bfloat16)
        k = lax.dot_general(
            x_flat, wkf, (((1,), (1,)), ((), ())),
            preferred_element_type=jnp.float32).astype(jnp.bfloat16)
        qr = (q * cosf
              + jnp.dot(q, P, preferred_element_type=jnp.float32
                        ).astype(jnp.bfloat16) * sinf)
        kr = (k * cosf
              + jnp.dot(k, P, preferred_element_type=jnp.float32
                        ).astype(jnp.bfloat16) * sinf)

        if h > 0:
            for r in range(2):
                make_rdma(r, h - 1, 1).wait_recv()
        if 0 < h < N_DEV - 1:
            for r in range(2):
                rdma = make_rdma(r, h, 1)
                rdma.start()
                started.append(rdma)

        wvf = comm[h, 1, 0]
        wof = comm[h, 1, 1]
        v = lax.dot_general(
            x_flat, wvf, (((1,), (1,)), ((), ())),
            preferred_element_type=jnp.float32).astype(jnp.bfloat16)
        for b in range(B_LOC):
            rsl = slice(b * SQ, (b + 1) * SQ)
            for hh in range(HQ_LOC):
                csl = slice(hh * DH, (hh + 1) * DH)
                s = lax.dot_general(
                    qr[rsl, csl], kr[rsl, csl], (((1,), (1,)), ((), ())),
                    preferred_element_type=jnp.float32)
                e = jnp.exp(s)
                w = e * (1.0 / jnp.sum(e, axis=1, keepdims=True))
                ctx_buf[rsl, csl] = jnp.dot(
                    w.astype(jnp.bfloat16), v[rsl, csl],
                    preferred_element_type=jnp.float32).astype(jnp.bfloat16)
        contrib = jnp.dot(ctx_buf[...], wof,
                          preferred_element_type=jnp.float32
                          ).reshape(B_LOC, SQ, D)
        if h == 0:
            out_ref[...] = contrib
        else:
            out_ref[...] = out_ref[...] + contrib

    for rdma in started:
        rdma.wait_send()


def kernel(x, Wq, Wk, Wv, Wo):
    cos_in = jnp.asarray(_COSF, dtype=jnp.bfloat16)
    sin_in = jnp.asarray(_SINF, dtype=jnp.bfloat16)
    p_in = jnp.asarray(_P, dtype=jnp.bfloat16)
    wqt = jnp.swapaxes(Wq, 0, 1)
    wkt = jnp.swapaxes(Wk, 0, 1)
    wvt = jnp.swapaxes(Wv, 0, 1)
    return pl.pallas_call(
        _body,
        out_shape=jax.ShapeDtypeStruct((B_LOC, SQ, D), jnp.float32),
        in_specs=[pl.BlockSpec(memory_space=pltpu.VMEM)] * 8,
        out_specs=pl.BlockSpec(memory_space=pltpu.VMEM),
        scratch_shapes=[
            pltpu.VMEM((BS, D), jnp.bfloat16),
            pltpu.VMEM((BS, PW), jnp.bfloat16),
            pltpu.VMEM((N_DEV, 2, 2, PW, D), jnp.bfloat16),
            pltpu.SemaphoreType.DMA((2, N_DEV - 1, 2)),
            pltpu.SemaphoreType.DMA((2, N_DEV - 1, 2)),
        ],
        compiler_params=pltpu.CompilerParams(collective_id=0),
    )(x, wqt, wkt, wvt, Wo, cos_in, sin_in, p_in)
